# baseline (speedup 1.0000x reference)
import jax
import jax.numpy as jnp
from jax import lax
from jax.experimental import pallas as pl
from jax.experimental.pallas import tpu as pltpu
from jax.experimental.pallas import tpu_sc as plsc

_N_IDS = 8192
_N_BINS = 256
_N_SUBCORES = 16
_IDS_PER_SUB = _N_IDS // _N_SUBCORES
_BINS_PER_SUB = _N_BINS // _N_SUBCORES
_L = 16
_D = 64
_H = 64
_O = 8


def _fused_body(ids_hbm, emb_hbm, w1_hbm, b1_hbm, w2_hbm, b2_hbm, idx0_hbm,
                out_hbm,
                ids_v, cnt_v, cnt16_v, emb_v, eloc_v, e_v, h_v,
                w1_v, w2_v, b1_v, b2_v, v_v, idx0_v,
                shared_cnt, shared_e):
    s = lax.axis_index("s")
    zeros16 = jnp.zeros((_L,), jnp.float32)

    pltpu.sync_copy(ids_hbm.at[pl.ds(s * _IDS_PER_SUB, _IDS_PER_SUB)], ids_v)
    pltpu.sync_copy(emb_hbm.at[pl.ds(s * _BINS_PER_SUB * _D, _BINS_PER_SUB * _D)],
                    emb_v)

    for j in range(_N_BINS // _L):
        cnt_v[0, pl.ds(j * _L, _L)] = zeros16
    ones = jnp.ones((_L,), jnp.float32)
    row0 = jnp.zeros((_L,), jnp.int32)
    for j in range(_IDS_PER_SUB // _L):
        idx = ids_v[pl.ds(j * _L, _L)]
        plsc.addupdate_scatter(cnt_v, [row0, idx], ones)

    pltpu.sync_copy(idx0_hbm, idx0_v)

    @pl.when(s == 0)
    def _():
        for c in range(_D // _L):
            eloc_v[0, pl.ds(c * _L, _L)] = zeros16
        pltpu.sync_copy(eloc_v, shared_e)
        for r in range(4):
            pltpu.sync_copy(eloc_v.at[0], shared_cnt.at[0, pl.ds(r * _D, _D)])

    plsc.subcore_barrier()

    pltpu.sync_copy(cnt_v, shared_cnt.at[idx0_v], add=True)
    plsc.subcore_barrier()

    pltpu.sync_copy(shared_cnt.at[0, pl.ds(s * _BINS_PER_SUB, _BINS_PER_SUB)],
                    cnt16_v)
    cnt16 = cnt16_v[pl.ds(0, _L)] * (1.0 / _N_IDS)
    acc = [zeros16 for _ in range(_D // _L)]
    for j in range(_BINS_PER_SUB):
        c = cnt16[j]
        for cc in range(_D // _L):
            acc[cc] = acc[cc] + c * emb_v[pl.ds(j * _D + cc * _L, _L)]
    for cc in range(_D // _L):
        eloc_v[0, pl.ds(cc * _L, _L)] = acc[cc]
    pltpu.sync_copy(eloc_v, shared_e.at[idx0_v], add=True)
    plsc.subcore_barrier()

    @pl.when(s == 0)
    def _():
        pltpu.sync_copy(shared_e, e_v)
        pltpu.sync_copy(w1_hbm, w1_v)
        pltpu.sync_copy(w2_hbm, w2_v.at[pl.ds(0, _O * _H)])
        pltpu.sync_copy(b1_hbm, b1_v)
        b2_v[pl.ds(0, _L)] = zeros16
        pltpu.sync_copy(b2_hbm, b2_v.at[pl.ds(0, _O)])
        for j in range(_O * _H, _L * _H, _L):
            w2_v[pl.ds(j, _L)] = zeros16

        lane = lax.broadcasted_iota(jnp.int32, (_L,), 0)
        e_blk = [e_v[0, pl.ds(kb * _L, _L)] for kb in range(_D // _L)]
        hs = []
        for jb in range(_H // _L):
            hj = b1_v[pl.ds(jb * _L, _L)]
            col = lane * _H + jb * _L * _H
            for k in range(_D):
                ek = e_blk[k // _L][k % _L]
                hj = hj + ek * plsc.load_gather(w1_v, [col + k])
            hs.append(jnp.maximum(hj, 0.0))

        v = b2_v[pl.ds(0, _L)]
        colw2 = lane * _H
        for k in range(_H):
            hk = hs[k // _L][k % _L]
            v = v + hk * plsc.load_gather(w2_v, [colw2 + k])
        v_v[pl.ds(0, _L)] = v
        pltpu.sync_copy(v_v, out_hbm)


_fused = pl.kernel(
    _fused_body,
    mesh=plsc.VectorSubcoreMesh(core_axis_name="c", subcore_axis_name="s",
                                num_cores=1),
    out_type=jax.ShapeDtypeStruct((_L,), jnp.float32),
    scratch_types=[
        pltpu.VMEM((_IDS_PER_SUB,), jnp.int32),
        pltpu.VMEM((1, _N_BINS), jnp.float32),
        pltpu.VMEM((_BINS_PER_SUB,), jnp.float32),
        pltpu.VMEM((_BINS_PER_SUB * _D,), jnp.float32),
        pltpu.VMEM((1, _D), jnp.float32),
        pltpu.VMEM((1, _D), jnp.float32),
        pltpu.VMEM((_H,), jnp.float32),
        pltpu.VMEM((_H * _D,), jnp.float32),
        pltpu.VMEM((_L * _H,), jnp.float32),
        pltpu.VMEM((_H,), jnp.float32),
        pltpu.VMEM((_L,), jnp.float32),
        pltpu.VMEM((_L,), jnp.float32),
        pltpu.VMEM((1,), jnp.int32),
        pltpu.VMEM_SHARED((1, _N_BINS), jnp.float32),
        pltpu.VMEM_SHARED((1, _D), jnp.float32),
    ],
    compiler_params=pltpu.CompilerParams(needs_layout_passes=False),
)


def kernel(ids, emb, W1, b1, W2, b2):
    ids32 = ids.astype(jnp.int32)
    out16 = _fused(ids32, emb.reshape(-1), W1.reshape(-1), b1,
                   W2.reshape(-1), b2, jnp.zeros((1,), jnp.int32))
    return out16[:_O]


# speedup vs baseline: 1.1504x; 1.1504x over previous
import jax
import jax.numpy as jnp
from jax import lax
from jax.experimental import pallas as pl
from jax.experimental.pallas import tpu as pltpu
from jax.experimental.pallas import tpu_sc as plsc

_N_IDS = 8192
_N_BINS = 256
_N_SUB = 16
_IDS_PER_SUB = _N_IDS // _N_SUB
_BINS_PER_SUB = _N_BINS // _N_SUB
_L = 16
_D = 64
_H = 64
_O = 8


def _fused_body(ids_hbm, emb_hbm, w1_hbm, b1_hbm, w2_hbm, b2_hbm, out_hbm,
                ids_v, cnt_v, cnt16x16_v, emb_v, eloc_v, e16_v,
                w1_v, w2_v, b1_v, b2_v, v_v,
                cnt_slots, e_slots, sem_ids, sem_emb, sem_w):
    s = lax.axis_index("s")
    zeros16 = jnp.zeros((_L,), jnp.float32)

    cp_ids = pltpu.async_copy(
        ids_hbm.at[pl.ds(s * _IDS_PER_SUB, _IDS_PER_SUB)], ids_v, sem_ids)
    cp_emb = pltpu.async_copy(
        emb_hbm.at[pl.ds(s * _BINS_PER_SUB * _D, _BINS_PER_SUB * _D)],
        emb_v, sem_emb)

    @pl.when(s == 0)
    def _():
        b2_v[pl.ds(0, _L)] = zeros16
        for j in range(_O * _H, _L * _H, _L):
            w2_v[pl.ds(j, _L)] = zeros16
        pltpu.async_copy(w1_hbm, w1_v, sem_w)
        pltpu.async_copy(w2_hbm, w2_v.at[pl.ds(0, _O * _H)], sem_w)
        pltpu.async_copy(b1_hbm, b1_v, sem_w)
        pltpu.async_copy(b2_hbm, b2_v.at[pl.ds(0, _O)], sem_w)

    cp_ids.wait()
    for j in range(_N_BINS // _L):
        cnt_v[pl.ds(j * _L, _L)] = zeros16
    ones = jnp.ones((_L,), jnp.float32)
    for j in range(_IDS_PER_SUB // _L):
        idx = ids_v[pl.ds(j * _L, _L)]
        plsc.addupdate_scatter(cnt_v, [idx], ones)
    pltpu.sync_copy(cnt_v, cnt_slots.at[s])
    plsc.subcore_barrier()

    pltpu.sync_copy(cnt_slots.at[:, pl.ds(s * _BINS_PER_SUB, _BINS_PER_SUB)],
                    cnt16x16_v)
    cnt16 = cnt16x16_v[0, pl.ds(0, _L)]
    for r in range(1, _N_SUB):
        cnt16 = cnt16 + cnt16x16_v[r, pl.ds(0, _L)]
    cnt16 = cnt16 * (1.0 / _N_IDS)

    cp_emb.wait()
    acc = [zeros16 for _ in range(_D // _L)]
    for j in range(_BINS_PER_SUB):
        c = cnt16[j]
        for cc in range(_D // _L):
            acc[cc] = acc[cc] + c * emb_v[pl.ds(j * _D + cc * _L, _L)]
    for cc in range(_D // _L):
        eloc_v[pl.ds(cc * _L, _L)] = acc[cc]
    pltpu.sync_copy(eloc_v, e_slots.at[s])
    plsc.subcore_barrier()

    @pl.when(s == 0)
    def _():
        pltpu.sync_copy(e_slots, e16_v)
        e_blk = []
        for cc in range(_D // _L):
            eb = e16_v[0, pl.ds(cc * _L, _L)]
            for r in range(1, _N_SUB):
                eb = eb + e16_v[r, pl.ds(cc * _L, _L)]
            e_blk.append(eb)

        pltpu.make_async_copy(w1_hbm, w1_v, sem_w).wait()
        pltpu.make_async_copy(w2_hbm, w2_v.at[pl.ds(0, _O * _H)], sem_w).wait()
        pltpu.make_async_copy(b1_hbm, b1_v, sem_w).wait()
        pltpu.make_async_copy(b2_hbm, b2_v.at[pl.ds(0, _O)], sem_w).wait()

        lane = lax.broadcasted_iota(jnp.int32, (_L,), 0)
        hs = []
        for jb in range(_H // _L):
            hj = b1_v[pl.ds(jb * _L, _L)]
            col = lane * _H + jb * _L * _H
            for k in range(_D):
                ek = e_blk[k // _L][k % _L]
                hj = hj + ek * plsc.load_gather(w1_v, [col + k])
            hs.append(jnp.maximum(hj, 0.0))

        v = b2_v[pl.ds(0, _L)]
        colw2 = lane * _H
        for k in range(_H):
            hk = hs[k // _L][k % _L]
            v = v + hk * plsc.load_gather(w2_v, [colw2 + k])
        v_v[pl.ds(0, _L)] = v
        pltpu.sync_copy(v_v, out_hbm)


_fused = pl.kernel(
    _fused_body,
    mesh=plsc.VectorSubcoreMesh(core_axis_name="c", subcore_axis_name="s",
                                num_cores=1),
    out_type=jax.ShapeDtypeStruct((_L,), jnp.float32),
    scratch_types=[
        pltpu.VMEM((_IDS_PER_SUB,), jnp.int32),
        pltpu.VMEM((_N_BINS,), jnp.float32),
        pltpu.VMEM((_N_SUB, _BINS_PER_SUB), jnp.float32),
        pltpu.VMEM((_BINS_PER_SUB * _D,), jnp.float32),
        pltpu.VMEM((_D,), jnp.float32),
        pltpu.VMEM((_N_SUB, _D), jnp.float32),
        pltpu.VMEM((_H * _D,), jnp.float32),
        pltpu.VMEM((_L * _H,), jnp.float32),
        pltpu.VMEM((_H,), jnp.float32),
        pltpu.VMEM((_L,), jnp.float32),
        pltpu.VMEM((_L,), jnp.float32),
        pltpu.VMEM_SHARED((_N_SUB, _N_BINS), jnp.float32),
        pltpu.VMEM_SHARED((_N_SUB, _D), jnp.float32),
        pltpu.SemaphoreType.DMA,
        pltpu.SemaphoreType.DMA,
        pltpu.SemaphoreType.DMA,
    ],
    compiler_params=pltpu.CompilerParams(needs_layout_passes=False,
                                         use_tc_tiling_on_sc=False),
)


def kernel(ids, emb, W1, b1, W2, b2):
    ids32 = ids.astype(jnp.int32)
    out16 = _fused(ids32, emb.reshape(-1), W1.reshape(-1), b1,
                   W2.reshape(-1), b2)
    return out16[:_O]
